# Initial kernel scaffold; baseline (speedup 1.0000x reference)
import jax
import jax.numpy as jnp
from jax.experimental import pallas as pl


def kernel(x, edge_index, edge_weight, W1, b1, W2, b2):
    raise NotImplementedError("write your pallas kernel here")


# speedup vs baseline: 10.5715x; 10.5715x over previous
import functools

import jax
import jax.numpy as jnp
from jax import lax
from jax.experimental import pallas as pl
from jax.experimental.pallas import tpu as pltpu
from jax.experimental.pallas import tpu_sc as plsc

N_NODES = 10000
N_PAD = 10240
IN_CH, HID_CH, OUT_CH = 128, 64, 64
K = 10
ALPHA = 0.1

NC, NS = 2, 16
NW = NC * NS
CHUNK = 128
NCH = 82
E_PER_W = NCH * CHUNK
E_PAD = NW * E_PER_W
ROWS_PER_TILE = N_NODES // NS

_mesh = plsc.VectorSubcoreMesh(
    core_axis_name="c", subcore_axis_name="s", num_cores=NC, num_subcores=NS)
_sc_params = pltpu.CompilerParams(
    needs_layout_passes=False, use_tc_tiling_on_sc=False)


@functools.partial(
    pl.kernel,
    out_type=jax.ShapeDtypeStruct((NC, N_PAD), jnp.float32),
    mesh=_mesh,
    compiler_params=_sc_params,
    scratch_types=[
        pltpu.VMEM((NCH, CHUNK), jnp.int32),
        pltpu.VMEM((NCH, CHUNK), jnp.float32),
        pltpu.VMEM((N_PAD,), jnp.float32),
        pltpu.VMEM_SHARED((N_PAD,), jnp.float32),
    ],
)
def _deg_kernel(col_hbm, w_hbm, degp_hbm, col_v, w_v, bounce_v, deg_sh):
    cid = lax.axis_index("c")
    sid = lax.axis_index("s")
    wid = sid * NC + cid
    pltpu.sync_copy(col_hbm.at[wid], col_v)
    pltpu.sync_copy(w_hbm.at[wid], w_v)

    zero16 = jnp.zeros((16,), jnp.float32)

    def _zero(i, carry):
        bounce_v[pl.ds(i * 16, 16)] = zero16
        return carry

    lax.fori_loop(0, N_PAD // 16, _zero, 0)

    @pl.when(sid == 0)
    def _():
        pltpu.sync_copy(bounce_v, deg_sh)

    plsc.subcore_barrier()

    def _scatter(j, carry):
        pltpu.sync_copy(w_v.at[j], deg_sh.at[col_v.at[j]], add=True)
        return carry

    lax.fori_loop(0, NCH, _scatter, 0)
    plsc.subcore_barrier()

    @pl.when(sid == 0)
    def _():
        pltpu.sync_copy(deg_sh, bounce_v)
        pltpu.sync_copy(bounce_v, degp_hbm.at[cid])


@functools.partial(
    pl.kernel,
    out_type=jax.ShapeDtypeStruct((NW, E_PER_W), jnp.float32),
    mesh=_mesh,
    compiler_params=_sc_params,
    scratch_types=[
        pltpu.VMEM((N_PAD,), jnp.float32),
        pltpu.VMEM((E_PER_W,), jnp.int32),
        pltpu.VMEM((E_PER_W,), jnp.int32),
        pltpu.VMEM((E_PER_W,), jnp.float32),
        pltpu.VMEM((E_PER_W,), jnp.float32),
    ],
)
def _norm_kernel(rowf, colf, wf, dinv_hbm, normf, dinv_v, row_v, col_v, w_v,
                 norm_v):
    cid = lax.axis_index("c")
    sid = lax.axis_index("s")
    wid = sid * NC + cid
    pltpu.sync_copy(dinv_hbm, dinv_v)
    pltpu.sync_copy(rowf.at[wid], row_v)
    pltpu.sync_copy(colf.at[wid], col_v)
    pltpu.sync_copy(wf.at[wid], w_v)

    def _body(g, carry):
        r16 = row_v[pl.ds(g * 16, 16)]
        c16 = col_v[pl.ds(g * 16, 16)]
        w16 = w_v[pl.ds(g * 16, 16)]
        dr = plsc.load_gather(dinv_v, [r16])
        dc = plsc.load_gather(dinv_v, [c16])
        norm_v[pl.ds(g * 16, 16)] = dr * w16 * dc
        return carry

    lax.fori_loop(0, E_PER_W // 16, _body, 0)
    pltpu.sync_copy(norm_v, normf.at[wid])


def _scale_chunk(buf, norm_v, j):
    for g in range(CHUNK // 16):
        n16 = norm_v[pl.ds(j * CHUNK + g * 16, 16)]
        for e in range(16):
            ne = jnp.broadcast_to(n16[e], (16,))
            r = g * 16 + e
            for f in range(OUT_CH // 16):
                buf[r, pl.ds(f * 16, 16)] = buf[r, pl.ds(f * 16, 16)] * ne


@functools.partial(
    pl.kernel,
    out_type=jax.ShapeDtypeStruct((NC, N_NODES, OUT_CH), jnp.float32),
    mesh=_mesh,
    compiler_params=_sc_params,
    scratch_types=[
        pltpu.VMEM((NCH, CHUNK), jnp.int32),
        pltpu.VMEM((NCH, CHUNK), jnp.int32),
        pltpu.VMEM((E_PER_W,), jnp.float32),
        pltpu.VMEM((CHUNK, OUT_CH), jnp.float32),
        pltpu.VMEM((CHUNK, OUT_CH), jnp.float32),
        pltpu.VMEM_SHARED((N_NODES, OUT_CH), jnp.float32),
        pltpu.SemaphoreType.DMA,
        pltpu.SemaphoreType.DMA,
        pltpu.SemaphoreType.DMA,
        pltpu.SemaphoreType.DMA,
    ],
)
def _round_kernel(row3, col3, normf, h_hbm, p_hbm, row_v, col_v, norm_v,
                  gb0, gb1, agg_sh, gsem0, gsem1, ssem0, ssem1):
    cid = lax.axis_index("c")
    sid = lax.axis_index("s")
    wid = sid * NC + cid
    pltpu.sync_copy(row3.at[wid], row_v)
    pltpu.sync_copy(col3.at[wid], col_v)
    pltpu.sync_copy(normf.at[wid], norm_v)

    zero16 = jnp.zeros((16,), jnp.float32)

    def _zero(i, carry):
        for f in range(OUT_CH // 16):
            gb0[i, pl.ds(f * 16, 16)] = zero16
        return carry

    lax.fori_loop(0, CHUNK, _zero, 0)
    base = sid * ROWS_PER_TILE
    for t in range(ROWS_PER_TILE // CHUNK):
        pltpu.sync_copy(gb0, agg_sh.at[pl.ds(base + t * CHUNK, CHUNK)])
    rem = ROWS_PER_TILE % CHUNK
    if rem:
        pltpu.sync_copy(
            gb0.at[pl.ds(0, rem)],
            agg_sh.at[pl.ds(base + (ROWS_PER_TILE // CHUNK) * CHUNK, rem)])
    plsc.subcore_barrier()

    pltpu.async_copy(h_hbm.at[row_v.at[0]], gb0, gsem0)
    pltpu.async_copy(h_hbm.at[row_v.at[1]], gb1, gsem1)

    def _iter(t, carry):
        j0 = 2 * t
        j1 = 2 * t + 1
        pltpu.make_async_copy(h_hbm.at[row_v.at[j0]], gb0, gsem0).wait()
        _scale_chunk(gb0, norm_v, j0)
        pltpu.async_copy(gb0, agg_sh.at[col_v.at[j0]], ssem0, add=True)
        pltpu.make_async_copy(h_hbm.at[row_v.at[j1]], gb1, gsem1).wait()
        _scale_chunk(gb1, norm_v, j1)
        pltpu.async_copy(gb1, agg_sh.at[col_v.at[j1]], ssem1, add=True)
        pltpu.make_async_copy(gb0, agg_sh.at[col_v.at[j0]], ssem0).wait()

        @pl.when(j0 + 2 < NCH)
        def _():
            pltpu.async_copy(h_hbm.at[row_v.at[j0 + 2]], gb0, gsem0)

        pltpu.make_async_copy(gb1, agg_sh.at[col_v.at[j1]], ssem1).wait()

        @pl.when(j1 + 2 < NCH)
        def _():
            pltpu.async_copy(h_hbm.at[row_v.at[j1 + 2]], gb1, gsem1)

        return carry

    lax.fori_loop(0, NCH // 2, _iter, 0)
    plsc.subcore_barrier()

    for t in range(ROWS_PER_TILE // CHUNK):
        pltpu.sync_copy(agg_sh.at[pl.ds(base + t * CHUNK, CHUNK)], gb0)
        pltpu.sync_copy(gb0, p_hbm.at[cid, pl.ds(base + t * CHUNK, CHUNK)])
    if rem:
        off = base + (ROWS_PER_TILE // CHUNK) * CHUNK
        pltpu.sync_copy(agg_sh.at[pl.ds(off, rem)], gb0.at[pl.ds(0, rem)])
        pltpu.sync_copy(gb0.at[pl.ds(0, rem)], p_hbm.at[cid, pl.ds(off, rem)])


def _mlp_body(x_ref, w1_ref, b1_ref, w2_ref, b2_ref, o_ref):
    h = jnp.dot(x_ref[...], w1_ref[...], preferred_element_type=jnp.float32)
    h = jnp.maximum(h + b1_ref[...], 0.0)
    h = jnp.dot(h, w2_ref[...], preferred_element_type=jnp.float32)
    o_ref[...] = h + b2_ref[...]


def _mlp(x, W1, b1, W2, b2):
    blk = 1000
    return pl.pallas_call(
        _mlp_body,
        grid=(N_NODES // blk,),
        in_specs=[
            pl.BlockSpec((blk, IN_CH), lambda i: (i, 0)),
            pl.BlockSpec((IN_CH, HID_CH), lambda i: (0, 0)),
            pl.BlockSpec((1, HID_CH), lambda i: (0, 0)),
            pl.BlockSpec((HID_CH, OUT_CH), lambda i: (0, 0)),
            pl.BlockSpec((1, OUT_CH), lambda i: (0, 0)),
        ],
        out_specs=pl.BlockSpec((blk, OUT_CH), lambda i: (i, 0)),
        out_shape=jax.ShapeDtypeStruct((N_NODES, OUT_CH), jnp.float32),
    )(x, W1, b1.reshape(1, HID_CH), W2, b2.reshape(1, OUT_CH))


def _dinv_body(degp_ref, o_ref):
    deg = degp_ref[0] + degp_ref[1]
    safe = jnp.where(deg > 0, deg, 1.0)
    o_ref[...] = jnp.where(deg > 0, lax.rsqrt(safe), 0.0)


def _dinv(degp):
    return pl.pallas_call(
        _dinv_body,
        out_shape=jax.ShapeDtypeStruct((N_PAD // 128, 128), jnp.float32),
    )(degp.reshape(NC, N_PAD // 128, 128))


def _combine_body(p_ref, h0_ref, o_ref):
    agg = p_ref[0] + p_ref[1]
    o_ref[...] = (1.0 - ALPHA) * agg + ALPHA * h0_ref[...]


def _combine(p, h0):
    blk = 1000
    return pl.pallas_call(
        _combine_body,
        grid=(N_NODES // blk,),
        in_specs=[
            pl.BlockSpec((NC, blk, OUT_CH), lambda i: (0, i, 0)),
            pl.BlockSpec((blk, OUT_CH), lambda i: (i, 0)),
        ],
        out_specs=pl.BlockSpec((blk, OUT_CH), lambda i: (i, 0)),
        out_shape=jax.ShapeDtypeStruct((N_NODES, OUT_CH), jnp.float32),
    )(p, h0)


def kernel(x, edge_index, edge_weight, W1, b1, W2, b2):
    pad = E_PAD - (edge_index.shape[1] + N_NODES)
    loop = jnp.arange(N_NODES, dtype=jnp.int32)
    zpad_i = jnp.zeros((pad,), jnp.int32)
    row = jnp.concatenate([edge_index[0].astype(jnp.int32), loop, zpad_i])
    col = jnp.concatenate([edge_index[1].astype(jnp.int32), loop, zpad_i])
    w = jnp.concatenate([
        edge_weight.astype(jnp.float32),
        jnp.ones((N_NODES,), jnp.float32),
        jnp.zeros((pad,), jnp.float32),
    ])
    row3 = row.reshape(NW, NCH, CHUNK)
    col3 = col.reshape(NW, NCH, CHUNK)
    w3 = w.reshape(NW, NCH, CHUNK)
    rowf = row.reshape(NW, E_PER_W)
    colf = col.reshape(NW, E_PER_W)
    wf = w.reshape(NW, E_PER_W)

    degp = _deg_kernel(col3, w3)
    dinv = _dinv(degp).reshape(N_PAD)
    normf = _norm_kernel(rowf, colf, wf, dinv)
    h0 = _mlp(x, W1, b1, W2, b2)

    h = h0
    for _ in range(K):
        p = _round_kernel(row3, col3, normf, h)
        h = _combine(p, h0)
    return h
